# baseline (speedup 1.0000x reference)
import functools

import jax
import jax.numpy as jnp
from jax import lax
from jax.experimental import pallas as pl
from jax.experimental.pallas import tpu as pltpu
from jax.experimental.pallas import tpu_sc as plsc

N_NODES = 10000
N_EDGES = 320000
D_IN = 128
D_HID = 256

NC = 2
NS = 16
N_WORKERS = NC * NS

CHUNK = 128
N_CHUNKS = 80
E_PAD = N_WORKERS * N_CHUNKS * CHUNK
N_PAD = 10240
ROWS_PER_TILE = N_PAD // NS
NBUF = 2


def _sc_agg(x, src_blk, dst_blk):
    mesh = plsc.VectorSubcoreMesh(core_axis_name="c", subcore_axis_name="s")

    @functools.partial(
        pl.kernel,
        out_type=jax.ShapeDtypeStruct((NC, N_PAD, D_IN), jnp.float32),
        mesh=mesh,
        scratch_types=[
            [pltpu.VMEM((CHUNK,), jnp.int32) for _ in range(NBUF)],
            [pltpu.VMEM((CHUNK,), jnp.int32) for _ in range(NBUF)],
            [pltpu.VMEM((CHUNK, D_IN), jnp.float32) for _ in range(NBUF)],
            pltpu.VMEM_SHARED((N_PAD, D_IN), jnp.float32),
            [pltpu.SemaphoreType.DMA for _ in range(NBUF)],
            [pltpu.SemaphoreType.DMA for _ in range(NBUF)],
            [pltpu.SemaphoreType.DMA for _ in range(NBUF)],
        ],
    )
    def k(x_hbm, src_hbm, dst_hbm, out_hbm, sbufs, dbufs, rows, agg_sh,
          isems, dsems, gsems):
        cid = lax.axis_index("c")
        sid = lax.axis_index("s")
        wid = sid * NC + cid

        def idx_start(j, b):
            pltpu.async_copy(src_hbm.at[wid, j], sbufs[b], isems[b])
            pltpu.async_copy(dst_hbm.at[wid, j], dbufs[b], dsems[b])

        def idx_wait(j, b):
            pltpu.make_async_copy(src_hbm.at[wid, j], sbufs[b],
                                  isems[b]).wait()

        def dst_wait(j, b):
            pltpu.make_async_copy(dst_hbm.at[wid, j], dbufs[b],
                                  dsems[b]).wait()

        def gather_start(b):
            pltpu.async_copy(x_hbm.at[sbufs[b]], rows[b], gsems[b])

        def gather_wait(b):
            pltpu.make_async_copy(x_hbm.at[sbufs[b]], rows[b],
                                  gsems[b]).wait()

        def zrow(r, carry):
            for c in range(D_IN // 16):
                rows[0][r, pl.ds(c * 16, 16)] = jnp.zeros((16,), jnp.float32)
            return carry
        lax.fori_loop(0, CHUNK, zrow, 0)
        nbase = sid * ROWS_PER_TILE
        for j in range(ROWS_PER_TILE // CHUNK):
            pltpu.sync_copy(rows[0], agg_sh.at[pl.ds(nbase + j * CHUNK, CHUNK)])

        plsc.subcore_barrier()

        def body(j, carry):
            pltpu.sync_copy(src_hbm.at[wid, j], sbufs[0])
            pltpu.sync_copy(dst_hbm.at[wid, j], dbufs[0])
            pltpu.async_copy(x_hbm.at[sbufs[0]], rows[0], gsems[0]).wait()
            pltpu.sync_copy(rows[0], agg_sh.at[dbufs[0]], add=True)
            return carry
        lax.fori_loop(0, N_CHUNKS, body, 0)

        plsc.subcore_barrier()
        pltpu.sync_copy(agg_sh.at[pl.ds(nbase, ROWS_PER_TILE)],
                        out_hbm.at[cid, pl.ds(nbase, ROWS_PER_TILE)])

    return k(x, src_blk, dst_blk)


BLK = 1000


def _mlp_body(x_ref, a0_ref, a1_ref, w1_ref, b1_ref, w2_ref, b2_ref, o_ref):
    feat = x_ref[...] + a0_ref[...] + a1_ref[...]
    h = lax.dot_general(feat, w1_ref[...], (((1,), (1,)), ((), ())),
                        preferred_element_type=jnp.float32)
    h = jnp.maximum(h + b1_ref[...], 0.0)
    o = lax.dot_general(h, w2_ref[...], (((1,), (1,)), ((), ())),
                        preferred_element_type=jnp.float32)
    o_ref[...] = o + b2_ref[...]


def _mlp(x, a0, a1, W1, b1, W2, b2):
    return pl.pallas_call(
        _mlp_body,
        grid=(N_NODES // BLK,),
        in_specs=[
            pl.BlockSpec((BLK, D_IN), lambda i: (i, 0)),
            pl.BlockSpec((BLK, D_IN), lambda i: (i, 0)),
            pl.BlockSpec((BLK, D_IN), lambda i: (i, 0)),
            pl.BlockSpec((D_HID, D_IN), lambda i: (0, 0)),
            pl.BlockSpec((1, D_HID), lambda i: (0, 0)),
            pl.BlockSpec((D_IN, D_HID), lambda i: (0, 0)),
            pl.BlockSpec((1, D_IN), lambda i: (0, 0)),
        ],
        out_specs=pl.BlockSpec((BLK, D_IN), lambda i: (i, 0)),
        out_shape=jax.ShapeDtypeStruct((N_NODES, D_IN), jnp.float32),
    )(x, a0, a1, W1, b1.reshape(1, D_HID), W2, b2.reshape(1, D_IN))


def kernel(x, edge_index, W1, b1, W2, b2):
    src = edge_index[0].astype(jnp.int32)
    dst = edge_index[1].astype(jnp.int32)
    n_extra = E_PAD - N_EDGES
    src_blk = jnp.concatenate(
        [src, jnp.zeros((n_extra,), jnp.int32)]).reshape(
            N_WORKERS, N_CHUNKS, CHUNK)
    dst_blk = jnp.concatenate(
        [dst, jnp.full((n_extra,), N_PAD - 1, jnp.int32)]).reshape(
            N_WORKERS, N_CHUNKS, CHUNK)
    agg = _sc_agg(x, src_blk, dst_blk)
    return _mlp(x, agg[0], agg[1], W1, b1.reshape(-1), W2, b2)


# speedup vs baseline: 2.1491x; 2.1491x over previous
import functools

import jax
import jax.numpy as jnp
from jax import lax
from jax.experimental import pallas as pl
from jax.experimental.pallas import tpu as pltpu
from jax.experimental.pallas import tpu_sc as plsc

N_NODES = 10000
N_EDGES = 320000
D_IN = 128
D_HID = 256

NC = 2
NS = 16
N_WORKERS = NC * NS

CHUNK = 80
N_CHUNKS = 126
EDGES_PER_TILE = N_CHUNKS * CHUNK
E_PAD = N_WORKERS * EDGES_PER_TILE
N_PAD = 10240
ROWS_PER_TILE = N_PAD // NS
ZROWS = 128
NBUF = 2


def _sc_agg(x, src, dst):
    mesh = plsc.VectorSubcoreMesh(core_axis_name="c", subcore_axis_name="s")

    @functools.partial(
        pl.kernel,
        out_type=jax.ShapeDtypeStruct((NC, N_PAD, D_IN), jnp.float32),
        mesh=mesh,
        scratch_types=[
            [pltpu.VMEM((CHUNK,), jnp.int32) for _ in range(NBUF)],
            [pltpu.VMEM((CHUNK,), jnp.int32) for _ in range(NBUF)],
            [pltpu.VMEM((CHUNK, D_IN), jnp.float32) for _ in range(NBUF)],
            pltpu.VMEM((ZROWS, D_IN), jnp.float32),
            pltpu.VMEM_SHARED((N_PAD, D_IN), jnp.float32),
            [pltpu.SemaphoreType.DMA for _ in range(NBUF)],
            [pltpu.SemaphoreType.DMA for _ in range(NBUF)],
            [pltpu.SemaphoreType.DMA for _ in range(NBUF)],
        ],
    )
    def k(x_hbm, src_hbm, dst_hbm, out_hbm, sbufs, dbufs, rows, zero_v,
          agg_sh, isems, dsems, gsems):
        cid = lax.axis_index("c")
        sid = lax.axis_index("s")
        wid = sid * NC + cid
        ebase = wid * EDGES_PER_TILE

        def idx_start(j, b):
            off = ebase + j * CHUNK
            pltpu.async_copy(src_hbm.at[pl.ds(off, CHUNK)], sbufs[b], isems[b])
            pltpu.async_copy(dst_hbm.at[pl.ds(off, CHUNK)], dbufs[b], dsems[b])

        def src_wait(j, b):
            off = ebase + j * CHUNK
            pltpu.make_async_copy(src_hbm.at[pl.ds(off, CHUNK)], sbufs[b],
                                  isems[b]).wait()

        def dst_wait(j, b):
            off = ebase + j * CHUNK
            pltpu.make_async_copy(dst_hbm.at[pl.ds(off, CHUNK)], dbufs[b],
                                  dsems[b]).wait()

        def gather_start(b):
            pltpu.async_copy(x_hbm.at[sbufs[b]], rows[b], gsems[b])

        def gather_wait(b):
            pltpu.make_async_copy(x_hbm.at[sbufs[b]], rows[b],
                                  gsems[b]).wait()

        idx_start(0, 0)
        idx_start(1, 1)

        def zrow(r, carry):
            for c in range(D_IN // 16):
                zero_v[r, pl.ds(c * 16, 16)] = jnp.zeros((16,), jnp.float32)
            return carry
        lax.fori_loop(0, ZROWS, zrow, 0)
        nbase = sid * ROWS_PER_TILE
        for j in range(ROWS_PER_TILE // ZROWS):
            pltpu.sync_copy(zero_v, agg_sh.at[pl.ds(nbase + j * ZROWS, ZROWS)])

        src_wait(0, 0)
        gather_start(0)

        plsc.subcore_barrier()

        def step(j, b, refill):
            src_wait(j + 1, b ^ 1)
            gather_start(b ^ 1)
            gather_wait(b)
            dst_wait(j, b)
            pltpu.sync_copy(rows[b], agg_sh.at[dbufs[b]], add=True)
            if refill:
                idx_start(j + 2, b)

        def body(t, carry):
            j0 = 2 * t
            step(j0, 0, True)
            step(j0 + 1, 1, True)
            return carry
        lax.fori_loop(0, N_CHUNKS // 2 - 1, body, 0)

        step(N_CHUNKS - 2, 0, False)
        gather_wait(1)
        dst_wait(N_CHUNKS - 1, 1)
        pltpu.sync_copy(rows[1], agg_sh.at[dbufs[1]], add=True)

        plsc.subcore_barrier()
        pltpu.sync_copy(agg_sh.at[pl.ds(nbase, ROWS_PER_TILE)],
                        out_hbm.at[cid, pl.ds(nbase, ROWS_PER_TILE)])

    return k(x, src, dst)


BLK = 1000


def _mlp_body(x_ref, a0_ref, a1_ref, w1_ref, b1_ref, w2_ref, b2_ref, o_ref):
    feat = x_ref[...] + a0_ref[...] + a1_ref[...]
    h = lax.dot_general(feat, w1_ref[...], (((1,), (1,)), ((), ())),
                        preferred_element_type=jnp.float32)
    h = jnp.maximum(h + b1_ref[...], 0.0)
    o = lax.dot_general(h, w2_ref[...], (((1,), (1,)), ((), ())),
                        preferred_element_type=jnp.float32)
    o_ref[...] = o + b2_ref[...]


def _mlp(x, a0, a1, W1, b1, W2, b2):
    return pl.pallas_call(
        _mlp_body,
        grid=(N_NODES // BLK,),
        in_specs=[
            pl.BlockSpec((BLK, D_IN), lambda i: (i, 0)),
            pl.BlockSpec((BLK, D_IN), lambda i: (i, 0)),
            pl.BlockSpec((BLK, D_IN), lambda i: (i, 0)),
            pl.BlockSpec((D_HID, D_IN), lambda i: (0, 0)),
            pl.BlockSpec((1, D_HID), lambda i: (0, 0)),
            pl.BlockSpec((D_IN, D_HID), lambda i: (0, 0)),
            pl.BlockSpec((1, D_IN), lambda i: (0, 0)),
        ],
        out_specs=pl.BlockSpec((BLK, D_IN), lambda i: (i, 0)),
        out_shape=jax.ShapeDtypeStruct((N_NODES, D_IN), jnp.float32),
    )(x, a0, a1, W1, b1.reshape(1, D_HID), W2, b2.reshape(1, D_IN))


def kernel(x, edge_index, W1, b1, W2, b2):
    src = edge_index[0].astype(jnp.int32)
    dst = edge_index[1].astype(jnp.int32)
    n_extra = E_PAD - N_EDGES
    src_p = jnp.concatenate([src, jnp.zeros((n_extra,), jnp.int32)])
    dst_p = jnp.concatenate([dst, jnp.full((n_extra,), N_PAD - 1, jnp.int32)])
    agg = _sc_agg(x, src_p, dst_p)
    return _mlp(x, agg[0], agg[1], W1, b1, W2, b2)


# speedup vs baseline: 2.2606x; 1.0519x over previous
import functools

import jax
import jax.numpy as jnp
from jax import lax
from jax.experimental import pallas as pl
from jax.experimental.pallas import tpu as pltpu
from jax.experimental.pallas import tpu_sc as plsc

N_NODES = 10000
N_EDGES = 320000
D_IN = 128
D_HID = 256

NC = 2
NS = 16
N_WORKERS = NC * NS

CHUNK = 80
N_CHUNKS = 126
EDGES_PER_TILE = N_CHUNKS * CHUNK
E_PAD = N_WORKERS * EDGES_PER_TILE
N_PAD = 10240
ROWS_PER_TILE = N_PAD // NS
ZROWS = 128
ND = 4


def _sc_agg(x, src, dst):
    mesh = plsc.VectorSubcoreMesh(core_axis_name="c", subcore_axis_name="s")

    @functools.partial(
        pl.kernel,
        out_type=jax.ShapeDtypeStruct((NC, N_PAD, D_IN), jnp.float32),
        mesh=mesh,
        scratch_types=[
            [pltpu.VMEM((CHUNK,), jnp.int32) for _ in range(2)],
            [pltpu.VMEM((CHUNK,), jnp.int32) for _ in range(ND)],
            [pltpu.VMEM((CHUNK, D_IN), jnp.float32) for _ in range(2)],
            pltpu.VMEM((ZROWS, D_IN), jnp.float32),
            pltpu.VMEM_SHARED((N_PAD, D_IN), jnp.float32),
            [pltpu.SemaphoreType.DMA for _ in range(2)],
            [pltpu.SemaphoreType.DMA for _ in range(ND)],
            [pltpu.SemaphoreType.DMA for _ in range(2)],
            [pltpu.SemaphoreType.DMA for _ in range(2)],
        ],
    )
    def k(x_hbm, src_hbm, dst_hbm, out_hbm, sbufs, dbufs, rows, zero_v,
          agg_sh, isems, dsems, gsems, ssems):
        cid = lax.axis_index("c")
        sid = lax.axis_index("s")
        wid = sid * NC + cid
        ebase = wid * EDGES_PER_TILE

        def src_start(j, b):
            off = ebase + j * CHUNK
            pltpu.async_copy(src_hbm.at[pl.ds(off, CHUNK)], sbufs[b], isems[b])

        def src_wait(j, b):
            off = ebase + j * CHUNK
            pltpu.make_async_copy(src_hbm.at[pl.ds(off, CHUNK)], sbufs[b],
                                  isems[b]).wait()

        def dst_start(j, u):
            off = ebase + j * CHUNK
            pltpu.async_copy(dst_hbm.at[pl.ds(off, CHUNK)], dbufs[u], dsems[u])

        def dst_wait(j, u):
            off = ebase + j * CHUNK
            pltpu.make_async_copy(dst_hbm.at[pl.ds(off, CHUNK)], dbufs[u],
                                  dsems[u]).wait()

        def gather_start(b):
            pltpu.async_copy(x_hbm.at[sbufs[b]], rows[b], gsems[b])

        def gather_wait(b):
            pltpu.make_async_copy(x_hbm.at[sbufs[b]], rows[b],
                                  gsems[b]).wait()

        def scatter_start(b, u):
            pltpu.async_copy(rows[b], agg_sh.at[dbufs[u]], ssems[b], add=True)

        def scatter_wait(b, u):
            pltpu.make_async_copy(rows[b], agg_sh.at[dbufs[u]],
                                  ssems[b]).wait()

        src_start(0, 0)
        src_start(1, 1)
        for u in range(ND):
            dst_start(u, u)

        def zrow(r, carry):
            for c in range(D_IN // 16):
                zero_v[r, pl.ds(c * 16, 16)] = jnp.zeros((16,), jnp.float32)
            return carry
        lax.fori_loop(0, ZROWS, zrow, 0)
        nbase = sid * ROWS_PER_TILE
        for j in range(ROWS_PER_TILE // ZROWS):
            pltpu.sync_copy(zero_v, agg_sh.at[pl.ds(nbase + j * ZROWS, ZROWS)])

        src_wait(0, 0)
        gather_start(0)

        plsc.subcore_barrier()

        def astep(j, b, u, first=False, with_next=True, src_refill=True,
                  dst_refill=True):
            if with_next:
                src_wait(j + 1, b ^ 1)
                if not first:
                    scatter_wait(b ^ 1, (u - 1) % 4)
                gather_start(b ^ 1)
            gather_wait(b)
            if src_refill:
                src_start(j + 2, b)
            dst_wait(j, u)
            scatter_start(b, u)
            if dst_refill:
                dst_start(j + 3, (u + 3) % 4)

        astep(0, 0, 0, first=True, dst_refill=False)

        def body(t, carry):
            j0 = 4 * t + 1
            for v in range(4):
                astep(j0 + v, (1 + v) % 2, (1 + v) % 4)
            return carry
        lax.fori_loop(0, 30, body, 0)

        astep(121, 1, 1)
        astep(122, 0, 2)
        astep(123, 1, 3, dst_refill=False)
        astep(124, 0, 0, src_refill=False, dst_refill=False)
        astep(125, 1, 1, with_next=False, src_refill=False, dst_refill=False)
        scatter_wait(0, 124 % 4)
        scatter_wait(1, 125 % 4)

        plsc.subcore_barrier()
        pltpu.sync_copy(agg_sh.at[pl.ds(nbase, ROWS_PER_TILE)],
                        out_hbm.at[cid, pl.ds(nbase, ROWS_PER_TILE)])

    return k(x, src, dst)


BLK = 1000


def _mlp_body(x_ref, a0_ref, a1_ref, w1_ref, b1_ref, w2_ref, b2_ref, o_ref):
    feat = x_ref[...] + a0_ref[...] + a1_ref[...]
    h = lax.dot_general(feat, w1_ref[...], (((1,), (1,)), ((), ())),
                        preferred_element_type=jnp.float32)
    h = jnp.maximum(h + b1_ref[...], 0.0)
    o = lax.dot_general(h, w2_ref[...], (((1,), (1,)), ((), ())),
                        preferred_element_type=jnp.float32)
    o_ref[...] = o + b2_ref[...]


def _mlp(x, a0, a1, W1, b1, W2, b2):
    return pl.pallas_call(
        _mlp_body,
        grid=(N_NODES // BLK,),
        in_specs=[
            pl.BlockSpec((BLK, D_IN), lambda i: (i, 0)),
            pl.BlockSpec((BLK, D_IN), lambda i: (i, 0)),
            pl.BlockSpec((BLK, D_IN), lambda i: (i, 0)),
            pl.BlockSpec((D_HID, D_IN), lambda i: (0, 0)),
            pl.BlockSpec((1, D_HID), lambda i: (0, 0)),
            pl.BlockSpec((D_IN, D_HID), lambda i: (0, 0)),
            pl.BlockSpec((1, D_IN), lambda i: (0, 0)),
        ],
        out_specs=pl.BlockSpec((BLK, D_IN), lambda i: (i, 0)),
        out_shape=jax.ShapeDtypeStruct((N_NODES, D_IN), jnp.float32),
    )(x, a0, a1, W1, b1.reshape(1, D_HID), W2, b2.reshape(1, D_IN))


def kernel(x, edge_index, W1, b1, W2, b2):
    src = edge_index[0].astype(jnp.int32)
    dst = edge_index[1].astype(jnp.int32)
    n_extra = E_PAD - N_EDGES
    src_p = jnp.concatenate([src, jnp.zeros((n_extra,), jnp.int32)])
    dst_p = jnp.concatenate([dst, jnp.full((n_extra,), N_PAD - 1, jnp.int32)])
    agg = _sc_agg(x, src_p, dst_p)
    return _mlp(x, agg[0], agg[1], W1, b1, W2, b2)


# speedup vs baseline: 2.3849x; 1.0550x over previous
import functools

import jax
import jax.numpy as jnp
from jax import lax
from jax.experimental import pallas as pl
from jax.experimental.pallas import tpu as pltpu
from jax.experimental.pallas import tpu_sc as plsc

N_NODES = 10000
N_EDGES = 320000
D_IN = 128
D_HID = 256

NC = 2
NS = 16
N_WORKERS = NC * NS

CHUNK = 80
N_CHUNKS = 126
EDGES_PER_TILE = N_CHUNKS * CHUNK
E_PAD = N_WORKERS * EDGES_PER_TILE
N_PAD = 10240
ROWS_PER_TILE = N_PAD // NS
NR = 4
NSRC = 4
NDST = 8


def _sc_agg(x, src, dst):
    mesh = plsc.VectorSubcoreMesh(core_axis_name="c", subcore_axis_name="s")

    @functools.partial(
        pl.kernel,
        out_type=jax.ShapeDtypeStruct((NC, N_PAD, D_IN), jnp.float32),
        mesh=mesh,
        scratch_types=[
            [pltpu.VMEM((CHUNK,), jnp.int32) for _ in range(NSRC)],
            [pltpu.VMEM((CHUNK,), jnp.int32) for _ in range(NDST)],
            [pltpu.VMEM((CHUNK, D_IN), jnp.float32) for _ in range(NR)],
            pltpu.VMEM_SHARED((N_PAD, D_IN), jnp.float32),
            [pltpu.SemaphoreType.DMA for _ in range(NSRC)],
            [pltpu.SemaphoreType.DMA for _ in range(NDST)],
            [pltpu.SemaphoreType.DMA for _ in range(NR)],
            [pltpu.SemaphoreType.DMA for _ in range(NR)],
        ],
    )
    def k(x_hbm, src_hbm, dst_hbm, out_hbm, sbufs, dbufs, rows, agg_sh,
          isems, dsems, gsems, ssems):
        cid = lax.axis_index("c")
        sid = lax.axis_index("s")
        wid = sid * NC + cid
        ebase = wid * EDGES_PER_TILE

        def src_start(j, b):
            off = ebase + j * CHUNK
            pltpu.async_copy(src_hbm.at[pl.ds(off, CHUNK)], sbufs[b], isems[b])

        def src_wait(j, b):
            off = ebase + j * CHUNK
            pltpu.make_async_copy(src_hbm.at[pl.ds(off, CHUNK)], sbufs[b],
                                  isems[b]).wait()

        def dst_start(j, u):
            off = ebase + j * CHUNK
            pltpu.async_copy(dst_hbm.at[pl.ds(off, CHUNK)], dbufs[u], dsems[u])

        def dst_wait(j, u):
            off = ebase + j * CHUNK
            pltpu.make_async_copy(dst_hbm.at[pl.ds(off, CHUNK)], dbufs[u],
                                  dsems[u]).wait()

        def gather_start(b):
            pltpu.async_copy(x_hbm.at[sbufs[b % NSRC]], rows[b], gsems[b])

        def gather_wait(b):
            pltpu.make_async_copy(x_hbm.at[sbufs[b % NSRC]], rows[b],
                                  gsems[b]).wait()

        def scatter_start(b, u):
            pltpu.async_copy(rows[b], agg_sh.at[dbufs[u]], ssems[b], add=True)

        def scatter_wait(b, u):
            pltpu.make_async_copy(rows[b], agg_sh.at[dbufs[u]],
                                  ssems[b]).wait()

        for b in range(NSRC):
            src_start(b, b)
        for u in range(NDST):
            dst_start(u, u)

        def zrow(r, carry):
            for c in range(D_IN // 16):
                rows[0][r, pl.ds(c * 16, 16)] = jnp.zeros((16,), jnp.float32)
            return carry
        lax.fori_loop(0, CHUNK, zrow, 0)
        nbase = sid * ROWS_PER_TILE
        for j in range(ROWS_PER_TILE // CHUNK):
            pltpu.sync_copy(rows[0], agg_sh.at[pl.ds(nbase + j * CHUNK, CHUNK)])

        for b in range(NR - 1):
            src_wait(b, b)
            gather_start(b)

        plsc.subcore_barrier()

        def astep(j, r, u, first=False, with_next=True, src_refill=True,
                  dst_refill=True):
            if with_next:
                src_wait(j + 3, (r + 3) % NSRC)
                if not first:
                    scatter_wait((r + 3) % NR, (u - 1) % NDST)
                gather_start((r + 3) % NR)
            gather_wait(r)
            if src_refill:
                src_start(j + 4, r % NSRC)
            dst_wait(j, u)
            scatter_start(r, u)
            if dst_refill:
                dst_start(j + 7, (u + 7) % NDST)

        astep(0, 0, 0, first=True, dst_refill=False)

        def body(t, carry):
            j0 = 8 * t + 1
            for v in range(8):
                astep(j0 + v, (1 + v) % NR, (1 + v) % NDST)
            return carry
        lax.fori_loop(0, 14, body, 0)

        for j in range(113, N_CHUNKS):
            astep(j, j % NR, j % NDST,
                  with_next=(j + 3 < N_CHUNKS),
                  src_refill=(j + 4 < N_CHUNKS),
                  dst_refill=(j + 7 < N_CHUNKS))
        for j in range(N_CHUNKS - NR, N_CHUNKS):
            scatter_wait(j % NR, j % NDST)

        plsc.subcore_barrier()
        pltpu.sync_copy(agg_sh.at[pl.ds(nbase, ROWS_PER_TILE)],
                        out_hbm.at[cid, pl.ds(nbase, ROWS_PER_TILE)])

    return k(x, src, dst)


BLK = 1000


def _mlp_body(x_ref, a0_ref, a1_ref, w1_ref, b1_ref, w2_ref, b2_ref, o_ref):
    feat = x_ref[...] + a0_ref[...] + a1_ref[...]
    h = lax.dot_general(feat, w1_ref[...], (((1,), (1,)), ((), ())),
                        preferred_element_type=jnp.float32)
    h = jnp.maximum(h + b1_ref[...], 0.0)
    o = lax.dot_general(h, w2_ref[...], (((1,), (1,)), ((), ())),
                        preferred_element_type=jnp.float32)
    o_ref[...] = o + b2_ref[...]


def _mlp(x, a0, a1, W1, b1, W2, b2):
    return pl.pallas_call(
        _mlp_body,
        grid=(N_NODES // BLK,),
        in_specs=[
            pl.BlockSpec((BLK, D_IN), lambda i: (i, 0)),
            pl.BlockSpec((BLK, D_IN), lambda i: (i, 0)),
            pl.BlockSpec((BLK, D_IN), lambda i: (i, 0)),
            pl.BlockSpec((D_HID, D_IN), lambda i: (0, 0)),
            pl.BlockSpec((1, D_HID), lambda i: (0, 0)),
            pl.BlockSpec((D_IN, D_HID), lambda i: (0, 0)),
            pl.BlockSpec((1, D_IN), lambda i: (0, 0)),
        ],
        out_specs=pl.BlockSpec((BLK, D_IN), lambda i: (i, 0)),
        out_shape=jax.ShapeDtypeStruct((N_NODES, D_IN), jnp.float32),
    )(x, a0, a1, W1, b1.reshape(1, D_HID), W2, b2.reshape(1, D_IN))


def kernel(x, edge_index, W1, b1, W2, b2):
    src = edge_index[0].astype(jnp.int32)
    dst = edge_index[1].astype(jnp.int32)
    n_extra = E_PAD - N_EDGES
    src_p = jnp.concatenate([src, jnp.zeros((n_extra,), jnp.int32)])
    dst_p = jnp.concatenate([dst, jnp.full((n_extra,), N_PAD - 1, jnp.int32)])
    agg = _sc_agg(x, src_p, dst_p)
    return _mlp(x, agg[0], agg[1], W1, b1, W2, b2)


# speedup vs baseline: 2.3852x; 1.0001x over previous
import functools

import jax
import jax.numpy as jnp
from jax import lax
from jax.experimental import pallas as pl
from jax.experimental.pallas import tpu as pltpu
from jax.experimental.pallas import tpu_sc as plsc

N_NODES = 10000
N_EDGES = 320000
D_IN = 128
D_HID = 256

NC = 2
NS = 16
N_WORKERS = NC * NS

CHUNK = 80
N_CHUNKS = 126
EDGES_PER_TILE = N_CHUNKS * CHUNK
E_PAD = N_WORKERS * EDGES_PER_TILE
N_PAD = 10240
ROWS_PER_TILE = N_PAD // NS
NR = 4
NSRC = 4
NDST = 8


def _sc_agg(x, src, dst):
    mesh = plsc.VectorSubcoreMesh(core_axis_name="c", subcore_axis_name="s")

    @functools.partial(
        pl.kernel,
        out_type=jax.ShapeDtypeStruct((NC, N_PAD, D_IN), jnp.float32),
        mesh=mesh,
        scratch_types=[
            [pltpu.VMEM((CHUNK,), jnp.int32) for _ in range(NSRC)],
            [pltpu.VMEM((CHUNK,), jnp.int32) for _ in range(NDST)],
            [pltpu.VMEM((CHUNK, D_IN), jnp.float32) for _ in range(NR)],
            pltpu.VMEM_SHARED((N_PAD, D_IN), jnp.float32),
            [pltpu.SemaphoreType.DMA for _ in range(NSRC)],
            [pltpu.SemaphoreType.DMA for _ in range(NDST)],
            [pltpu.SemaphoreType.DMA for _ in range(NR)],
            [pltpu.SemaphoreType.DMA for _ in range(NR)],
        ],
    )
    def k(x_hbm, src_hbm, dst_hbm, out_hbm, sbufs, dbufs, rows, agg_sh,
          isems, dsems, gsems, ssems):
        cid = lax.axis_index("c")
        sid = lax.axis_index("s")
        wid = sid * NC + cid
        ebase = wid * EDGES_PER_TILE

        def src_start(j, b):
            off = ebase + j * CHUNK
            pltpu.async_copy(src_hbm.at[pl.ds(off, CHUNK)], sbufs[b], isems[b])

        def src_wait(j, b):
            off = ebase + j * CHUNK
            pltpu.make_async_copy(src_hbm.at[pl.ds(off, CHUNK)], sbufs[b],
                                  isems[b]).wait()

        def dst_start(j, u):
            off = ebase + j * CHUNK
            pltpu.async_copy(dst_hbm.at[pl.ds(off, CHUNK)], dbufs[u], dsems[u])

        def dst_wait(j, u):
            off = ebase + j * CHUNK
            pltpu.make_async_copy(dst_hbm.at[pl.ds(off, CHUNK)], dbufs[u],
                                  dsems[u]).wait()

        def gather_start(b):
            pltpu.async_copy(x_hbm.at[sbufs[b % NSRC]], rows[b], gsems[b])

        def gather_wait(b):
            pltpu.make_async_copy(x_hbm.at[sbufs[b % NSRC]], rows[b],
                                  gsems[b]).wait()

        def scatter_start(b, u):
            pltpu.async_copy(rows[b], agg_sh.at[dbufs[u]], ssems[b], add=True)

        def scatter_wait(b, u):
            pltpu.make_async_copy(rows[b], agg_sh.at[dbufs[u]],
                                  ssems[b]).wait()

        for b in range(NSRC):
            src_start(b, b)
        for u in range(NDST):
            dst_start(u, u)

        def zrow(r, carry):
            for c in range(D_IN // 16):
                rows[0][r, pl.ds(c * 16, 16)] = jnp.zeros((16,), jnp.float32)
            return carry
        lax.fori_loop(0, CHUNK, zrow, 0)
        nbase = sid * ROWS_PER_TILE
        for j in range(ROWS_PER_TILE // CHUNK):
            pltpu.sync_copy(rows[0], agg_sh.at[pl.ds(nbase + j * CHUNK, CHUNK)])
        _rem = ROWS_PER_TILE % CHUNK
        if _rem:
            pltpu.sync_copy(
                rows[0].at[pl.ds(0, _rem)],
                agg_sh.at[pl.ds(nbase + ROWS_PER_TILE - _rem, _rem)])

        for b in range(NR - 1):
            src_wait(b, b)
            gather_start(b)

        plsc.subcore_barrier()

        def astep(j, r, u, first=False, with_next=True, src_refill=True,
                  dst_refill=True):
            if with_next:
                src_wait(j + 3, (r + 3) % NSRC)
                if not first:
                    scatter_wait((r + 3) % NR, (u - 1) % NDST)
                gather_start((r + 3) % NR)
            gather_wait(r)
            if src_refill:
                src_start(j + 4, r % NSRC)
            dst_wait(j, u)
            scatter_start(r, u)
            if dst_refill:
                dst_start(j + 7, (u + 7) % NDST)

        astep(0, 0, 0, first=True, dst_refill=False)

        n_body = (N_CHUNKS - 9) // 8
        tail_start = 1 + 8 * n_body

        def body(t, carry):
            j0 = 8 * t + 1
            for v in range(8):
                astep(j0 + v, (1 + v) % NR, (1 + v) % NDST)
            return carry
        lax.fori_loop(0, n_body, body, 0)

        for j in range(tail_start, N_CHUNKS):
            astep(j, j % NR, j % NDST,
                  with_next=(j + 3 < N_CHUNKS),
                  src_refill=(j + 4 < N_CHUNKS),
                  dst_refill=(j + 7 < N_CHUNKS))
        for j in range(N_CHUNKS - NR, N_CHUNKS):
            scatter_wait(j % NR, j % NDST)

        plsc.subcore_barrier()
        pltpu.sync_copy(agg_sh.at[pl.ds(nbase, ROWS_PER_TILE)],
                        out_hbm.at[cid, pl.ds(nbase, ROWS_PER_TILE)])

    return k(x, src, dst)


BLK = 1000


def _mlp_body(x_ref, a0_ref, a1_ref, w1_ref, b1_ref, w2_ref, b2_ref, o_ref):
    feat = x_ref[...] + a0_ref[...] + a1_ref[...]
    h = lax.dot_general(feat, w1_ref[...], (((1,), (1,)), ((), ())),
                        preferred_element_type=jnp.float32)
    h = jnp.maximum(h + b1_ref[...], 0.0)
    o = lax.dot_general(h, w2_ref[...], (((1,), (1,)), ((), ())),
                        preferred_element_type=jnp.float32)
    o_ref[...] = o + b2_ref[...]


def _mlp(x, a0, a1, W1, b1, W2, b2):
    return pl.pallas_call(
        _mlp_body,
        grid=(N_NODES // BLK,),
        in_specs=[
            pl.BlockSpec((BLK, D_IN), lambda i: (i, 0)),
            pl.BlockSpec((BLK, D_IN), lambda i: (i, 0)),
            pl.BlockSpec((BLK, D_IN), lambda i: (i, 0)),
            pl.BlockSpec((D_HID, D_IN), lambda i: (0, 0)),
            pl.BlockSpec((1, D_HID), lambda i: (0, 0)),
            pl.BlockSpec((D_IN, D_HID), lambda i: (0, 0)),
            pl.BlockSpec((1, D_IN), lambda i: (0, 0)),
        ],
        out_specs=pl.BlockSpec((BLK, D_IN), lambda i: (i, 0)),
        out_shape=jax.ShapeDtypeStruct((N_NODES, D_IN), jnp.float32),
    )(x, a0, a1, W1, b1.reshape(1, D_HID), W2, b2.reshape(1, D_IN))


def kernel(x, edge_index, W1, b1, W2, b2):
    src = edge_index[0].astype(jnp.int32)
    dst = edge_index[1].astype(jnp.int32)
    n_extra = E_PAD - N_EDGES
    src_p = jnp.concatenate([src, jnp.zeros((n_extra,), jnp.int32)])
    dst_p = jnp.concatenate([dst, jnp.full((n_extra,), N_PAD - 1, jnp.int32)])
    agg = _sc_agg(x, src_p, dst_p)
    return _mlp(x, agg[0], agg[1], W1, b1, W2, b2)
